# Initial kernel scaffold; baseline (speedup 1.0000x reference)
import jax
import jax.numpy as jnp
from jax.experimental import pallas as pl


def kernel(src_ids, etype, block_src, block_dst, e2t_ent, e2t_typ, t2c_typ, t2c_clu, e2c_ent, e2c_clu, entity_emb, type_emb, cluster_emb, relation, ln_g, ln_b, cl_w1, cl_b1, cl_w2, cl_b2, fc_w, fc_b, attn_a, gate_w, exp_w1, exp_w2):
    raise NotImplementedError("write your pallas kernel here")


# speedup vs baseline: 1.0566x; 1.0566x over previous
import jax
import jax.numpy as jnp
from jax import lax
from jax.experimental import pallas as pl

_NE = 50000; _NT = 512; _NC = 100; _NR = 400; _D = 128
_TAU = 0.5; _DECAY = 1e-4; _CLW = 1e-3; _NLAYERS = 2
_NSRC = 4096; _NDST = 1024; _EB = 32768
_H = 8; _DH = 64; _NEXP = 8; _DFF = 512
_BE = 2048; _NBLK = _EB // _BE
_NEG = -1e30


def _spmm(src, dst, n, x):
    s = jnp.concatenate([src, dst]); d = jnp.concatenate([dst, src])
    deg = jnp.maximum(jnp.bincount(s, length=n).astype(jnp.float32), 1.0)
    w = 1.0 / jnp.sqrt(deg[s] * deg[d])
    return jax.ops.segment_sum(w[:, None] * x[s], d, num_segments=n)


def _light_gcn(a, b, src, dst):
    nA = a.shape[0]; n = nA + b.shape[0]
    ego = jnp.concatenate([a, b], 0); acc = [ego]
    for _ in range(_NLAYERS):
        ego = _spmm(src, dst + nA, n, ego)
        acc.append(ego)
    m = jnp.mean(jnp.stack(acc, 1), 1)
    return m[:nA], m[nA:]


def _ln(x, g, b):
    mu = x.mean(-1, keepdims=True)
    v = ((x - mu) ** 2).mean(-1, keepdims=True)
    return (x - mu) / jnp.sqrt(v + 1e-5) * g + b


def _elu(x):
    return jnp.where(x > 0, x, jnp.expm1(jnp.minimum(x, 0.0)))


def _cl_proj(x, w1, b1, w2, b2):
    return _elu(x @ w1 + b1) @ w2 + b2


def _cl_loss(A, B, w1, b1, w2, b2):
    A = _cl_proj(A, w1, b1, w2, b2)
    B = _cl_proj(B, w1, b1, w2, b2)
    An = A / (jnp.linalg.norm(A, axis=1, keepdims=True) + 1e-9)
    Bn = B / (jnp.linalg.norm(B, axis=1, keepdims=True) + 1e-9)
    f = lambda m: jnp.exp(m / _TAU)
    r1 = f(An @ An.T); c1 = f(An @ Bn.T)
    l1 = -jnp.log(jnp.diag(c1) / (r1.sum(1) + c1.sum(1) - jnp.diag(r1)))
    r2 = f(Bn @ Bn.T); c2 = f(Bn @ An.T)
    l2 = -jnp.log(jnp.diag(c2) / (r2.sum(1) + c2.sum(1) - jnp.diag(r2)))
    return ((l1 + l2) * 0.5).mean()


def _head_mat():
    r = lax.broadcasted_iota(jnp.int32, (_H * _DH, _H), 0) // _DH
    c = lax.broadcasted_iota(jnp.int32, (_H * _DH, _H), 1)
    return (r == c).astype(jnp.float32)


def _expand_mat():
    r = lax.broadcasted_iota(jnp.int32, (_H, _H * _DH), 0)
    c = lax.broadcasted_iota(jnp.int32, (_H, _H * _DH), 1) // _DH
    return (r == c).astype(jnp.float32)


def _onehot(dst):
    ids = lax.broadcasted_iota(jnp.int32, (_BE, _NDST), 1)
    return ids == dst


def _k1(srcg_ref, rel_ref, fcw_ref, fcb_ref, attn_ref, p1_ref, e_ref):
    msg = jnp.maximum(srcg_ref[...] + rel_ref[...], 0.0)
    p1 = jnp.dot(msg, fcw_ref[...], preferred_element_type=jnp.float32)
    p1 = p1 + fcb_ref[...]
    p1_ref[...] = p1
    q = p1 * attn_ref[...]
    e_ref[...] = jnp.dot(q, _head_mat(), preferred_element_type=jnp.float32)


def _k2(e_ref, dst_ref, mx_ref):
    @pl.when(pl.program_id(0) == 0)
    def _():
        mx_ref[...] = jnp.full((_H, _NDST), _NEG, jnp.float32)

    e = e_ref[...]
    A = _onehot(dst_ref[...])
    for h in range(_H):
        eh = jnp.broadcast_to(e[:, h:h + 1], (_BE, _NDST))
        m = jnp.max(jnp.where(A, eh, _NEG), axis=0, keepdims=True)
        mx_ref[h:h + 1, :] = jnp.maximum(mx_ref[h:h + 1, :], m)


def _k3(p1_ref, e_ref, dst_ref, mx_ref, den_ref, num_ref):
    @pl.when(pl.program_id(0) == 0)
    def _():
        den_ref[...] = jnp.zeros((_NDST, _H), jnp.float32)
        num_ref[...] = jnp.zeros((_NDST, _NT), jnp.float32)

    A = _onehot(dst_ref[...]).astype(jnp.float32)
    mx_e = lax.dot_general(A, mx_ref[...], (((1,), (1,)), ((), ())),
                           preferred_element_type=jnp.float32)
    ex = jnp.exp(e_ref[...] - mx_e)
    den_ref[...] += lax.dot_general(A, ex, (((0,), (0,)), ((), ())),
                                    preferred_element_type=jnp.float32)
    exw = jnp.dot(ex, _expand_mat(), preferred_element_type=jnp.float32)
    w = exw * p1_ref[...]
    num_ref[...] += lax.dot_general(A, w, (((0,), (0,)), ((), ())),
                                    preferred_element_type=jnp.float32)


def _k4(num_ref, den_ref, gatew_ref, w1_ref, w2_ref, out_ref):
    den_e = jnp.dot(den_ref[...], _expand_mat(),
                    preferred_element_type=jnp.float32)
    pooled = num_ref[...] / (den_e + 1e-9)
    gl = jnp.dot(pooled, gatew_ref[...], preferred_element_type=jnp.float32)
    m1 = jnp.max(gl, axis=1, keepdims=True)
    g1 = gl == m1
    m2 = jnp.max(jnp.where(g1, _NEG, gl), axis=1, keepdims=True)
    g2 = jnp.logical_and(gl == m2, jnp.logical_not(g1))
    w1v = 1.0 / (1.0 + jnp.exp(m2 - m1))
    w2v = 1.0 - w1v
    gates = jnp.where(g1, w1v, 0.0) + jnp.where(g2, w2v, 0.0)
    acc = jnp.zeros((_NDST, _NT), jnp.float32)
    for ei in range(_NEXP):
        W1 = w1_ref[ei * _NT:(ei + 1) * _NT, :]
        h1 = jnp.maximum(
            jnp.dot(pooled, W1, preferred_element_type=jnp.float32), 0.0)
        W2 = w2_ref[ei * _DFF:(ei + 1) * _DFF, :]
        h2 = jnp.dot(h1, W2, preferred_element_type=jnp.float32)
        acc = acc + gates[:, ei:ei + 1] * h2
    out_ref[...] = 1.0 / (1.0 + jnp.exp(-acc))


def kernel(src_ids, etype, block_src, block_dst, e2t_ent, e2t_typ, t2c_typ,
           t2c_clu, e2c_ent, e2c_clu, entity_emb, type_emb, cluster_emb,
           relation, ln_g, ln_b, cl_w1, cl_b1, cl_w2, cl_b2, fc_w, fc_b,
           attn_a, gate_w, exp_w1, exp_w2):
    e2t_e, e2t_t = _light_gcn(entity_emb, type_emb, e2t_ent, e2t_typ)
    t2c_t, t2c_c = _light_gcn(type_emb, cluster_emb, t2c_typ, t2c_clu)
    e2c_e, e2c_c = _light_gcn(entity_emb, cluster_emb, e2c_ent, e2c_clu)
    ln = lambda x: _ln(x, ln_g, ln_b)
    node1 = jnp.concatenate([ln(e2t_e), ln(e2t_t), ln(t2c_c)], 0)
    node2 = jnp.concatenate([ln(e2c_e), ln(t2c_t), ln(e2c_c)], 0)
    src1 = node1[src_ids]; src2 = node2[src_ids]
    cl = _cl_loss(src1, src2, cl_w1, cl_b1, cl_w2, cl_b2)
    src = jnp.concatenate([src1, src2], -1)
    rel = relation[etype % _NR] * jnp.where(etype >= _NR, -1.0, 1.0)[:, None]
    reg = (jnp.sum(src ** 2) + jnp.sum(rel ** 2)) / 2.0
    aux = _CLW * cl + _DECAY * reg / _EB

    srcg = src[block_src]
    dst2 = block_dst.astype(jnp.int32).reshape(_EB, 1)
    fcb2 = fc_b.reshape(1, _NT)
    attn2 = attn_a.reshape(1, _H * _DH)

    p1, e = pl.pallas_call(
        _k1,
        grid=(_NBLK,),
        in_specs=[
            pl.BlockSpec((_BE, 2 * _D), lambda i: (i, 0)),
            pl.BlockSpec((_BE, 2 * _D), lambda i: (i, 0)),
            pl.BlockSpec((2 * _D, _NT), lambda i: (0, 0)),
            pl.BlockSpec((1, _NT), lambda i: (0, 0)),
            pl.BlockSpec((1, _H * _DH), lambda i: (0, 0)),
        ],
        out_specs=[
            pl.BlockSpec((_BE, _NT), lambda i: (i, 0)),
            pl.BlockSpec((_BE, _H), lambda i: (i, 0)),
        ],
        out_shape=[
            jax.ShapeDtypeStruct((_EB, _NT), jnp.float32),
            jax.ShapeDtypeStruct((_EB, _H), jnp.float32),
        ],
    )(srcg, rel, fc_w, fcb2, attn2)

    mx = pl.pallas_call(
        _k2,
        grid=(_NBLK,),
        in_specs=[
            pl.BlockSpec((_BE, _H), lambda i: (i, 0)),
            pl.BlockSpec((_BE, 1), lambda i: (i, 0)),
        ],
        out_specs=pl.BlockSpec((_H, _NDST), lambda i: (0, 0)),
        out_shape=jax.ShapeDtypeStruct((_H, _NDST), jnp.float32),
    )(e, dst2)

    den, num = pl.pallas_call(
        _k3,
        grid=(_NBLK,),
        in_specs=[
            pl.BlockSpec((_BE, _NT), lambda i: (i, 0)),
            pl.BlockSpec((_BE, _H), lambda i: (i, 0)),
            pl.BlockSpec((_BE, 1), lambda i: (i, 0)),
            pl.BlockSpec((_H, _NDST), lambda i: (0, 0)),
        ],
        out_specs=[
            pl.BlockSpec((_NDST, _H), lambda i: (0, 0)),
            pl.BlockSpec((_NDST, _NT), lambda i: (0, 0)),
        ],
        out_shape=[
            jax.ShapeDtypeStruct((_NDST, _H), jnp.float32),
            jax.ShapeDtypeStruct((_NDST, _NT), jnp.float32),
        ],
    )(p1, e, dst2, mx)

    predict = pl.pallas_call(
        _k4,
        grid=(1,),
        in_specs=[
            pl.BlockSpec((_NDST, _NT), lambda i: (0, 0)),
            pl.BlockSpec((_NDST, _H), lambda i: (0, 0)),
            pl.BlockSpec((_NT, _NEXP), lambda i: (0, 0)),
            pl.BlockSpec((_NEXP * _NT, _DFF), lambda i: (0, 0)),
            pl.BlockSpec((_NEXP * _DFF, _NT), lambda i: (0, 0)),
        ],
        out_specs=pl.BlockSpec((_NDST, _NT), lambda i: (0, 0)),
        out_shape=jax.ShapeDtypeStruct((_NDST, _NT), jnp.float32),
    )(num, den, gate_w, exp_w1.reshape(_NEXP * _NT, _DFF),
      exp_w2.reshape(_NEXP * _DFF, _NT))

    return predict, aux


# speedup vs baseline: 1.0567x; 1.0001x over previous
import jax
import jax.numpy as jnp
from jax import lax
from jax.experimental import pallas as pl

_NE = 50000; _NT = 512; _NC = 100; _NR = 400; _D = 128
_TAU = 0.5; _DECAY = 1e-4; _CLW = 1e-3; _NLAYERS = 2
_NSRC = 4096; _NDST = 1024; _EB = 32768
_H = 8; _DH = 64; _NEXP = 8; _DFF = 512
_BE = 4096; _NBLK = _EB // _BE
_NEG = -1e30


def _spmm(src, dst, n, x):
    s = jnp.concatenate([src, dst]); d = jnp.concatenate([dst, src])
    deg = jnp.maximum(jnp.bincount(s, length=n).astype(jnp.float32), 1.0)
    w = 1.0 / jnp.sqrt(deg[s] * deg[d])
    return jax.ops.segment_sum(w[:, None] * x[s], d, num_segments=n)


def _light_gcn(a, b, src, dst):
    nA = a.shape[0]; n = nA + b.shape[0]
    ego = jnp.concatenate([a, b], 0); acc = [ego]
    for _ in range(_NLAYERS):
        ego = _spmm(src, dst + nA, n, ego)
        acc.append(ego)
    m = jnp.mean(jnp.stack(acc, 1), 1)
    return m[:nA], m[nA:]


def _ln(x, g, b):
    mu = x.mean(-1, keepdims=True)
    v = ((x - mu) ** 2).mean(-1, keepdims=True)
    return (x - mu) / jnp.sqrt(v + 1e-5) * g + b


def _elu(x):
    return jnp.where(x > 0, x, jnp.expm1(jnp.minimum(x, 0.0)))


def _cl_proj(x, w1, b1, w2, b2):
    return _elu(x @ w1 + b1) @ w2 + b2


def _cl_loss(A, B, w1, b1, w2, b2):
    A = _cl_proj(A, w1, b1, w2, b2)
    B = _cl_proj(B, w1, b1, w2, b2)
    An = A / (jnp.linalg.norm(A, axis=1, keepdims=True) + 1e-9)
    Bn = B / (jnp.linalg.norm(B, axis=1, keepdims=True) + 1e-9)
    f = lambda m: jnp.exp(m / _TAU)
    r1 = f(An @ An.T); c1 = f(An @ Bn.T)
    l1 = -jnp.log(jnp.diag(c1) / (r1.sum(1) + c1.sum(1) - jnp.diag(r1)))
    r2 = f(Bn @ Bn.T); c2 = f(Bn @ An.T)
    l2 = -jnp.log(jnp.diag(c2) / (r2.sum(1) + c2.sum(1) - jnp.diag(r2)))
    return ((l1 + l2) * 0.5).mean()


def _head_mat():
    r = lax.broadcasted_iota(jnp.int32, (_H * _DH, _H), 0) // _DH
    c = lax.broadcasted_iota(jnp.int32, (_H * _DH, _H), 1)
    return (r == c).astype(jnp.float32)


def _expand_mat():
    r = lax.broadcasted_iota(jnp.int32, (_H, _H * _DH), 0)
    c = lax.broadcasted_iota(jnp.int32, (_H, _H * _DH), 1) // _DH
    return (r == c).astype(jnp.float32)


def _onehot(dst):
    ids = lax.broadcasted_iota(jnp.int32, (_BE, _NDST), 1)
    return ids == dst


def _k1(srcg_ref, rel_ref, fcw_ref, fcb_ref, attn_ref, p1_ref, e_ref):
    msg = jnp.maximum(srcg_ref[...] + rel_ref[...], 0.0)
    p1 = jnp.dot(msg, fcw_ref[...], preferred_element_type=jnp.float32)
    p1 = p1 + fcb_ref[...]
    p1_ref[...] = p1
    q = p1 * attn_ref[...]
    e_ref[...] = jnp.dot(q, _head_mat(), preferred_element_type=jnp.float32)


def _k2(e_ref, dst_ref, mx_ref):
    @pl.when(pl.program_id(0) == 0)
    def _():
        mx_ref[...] = jnp.full((_H, _NDST), _NEG, jnp.float32)

    e = e_ref[...]
    A = _onehot(dst_ref[...])
    for h in range(_H):
        eh = jnp.broadcast_to(e[:, h:h + 1], (_BE, _NDST))
        m = jnp.max(jnp.where(A, eh, _NEG), axis=0, keepdims=True)
        mx_ref[h:h + 1, :] = jnp.maximum(mx_ref[h:h + 1, :], m)


def _k3(p1_ref, e_ref, dst_ref, mx_ref, den_ref, num_ref):
    @pl.when(pl.program_id(0) == 0)
    def _():
        den_ref[...] = jnp.zeros((_NDST, _H), jnp.float32)
        num_ref[...] = jnp.zeros((_NDST, _NT), jnp.float32)

    A = _onehot(dst_ref[...]).astype(jnp.float32)
    mx_e = lax.dot_general(A, mx_ref[...], (((1,), (1,)), ((), ())),
                           preferred_element_type=jnp.float32)
    ex = jnp.exp(e_ref[...] - mx_e)
    den_ref[...] += lax.dot_general(A, ex, (((0,), (0,)), ((), ())),
                                    preferred_element_type=jnp.float32)
    exw = jnp.dot(ex, _expand_mat(), preferred_element_type=jnp.float32)
    w = exw * p1_ref[...]
    num_ref[...] += lax.dot_general(A, w, (((0,), (0,)), ((), ())),
                                    preferred_element_type=jnp.float32)


def _k4(num_ref, den_ref, gatew_ref, w1_ref, w2_ref, out_ref):
    den_e = jnp.dot(den_ref[...], _expand_mat(),
                    preferred_element_type=jnp.float32)
    pooled = num_ref[...] / (den_e + 1e-9)
    gl = jnp.dot(pooled, gatew_ref[...], preferred_element_type=jnp.float32)
    m1 = jnp.max(gl, axis=1, keepdims=True)
    g1 = gl == m1
    m2 = jnp.max(jnp.where(g1, _NEG, gl), axis=1, keepdims=True)
    g2 = jnp.logical_and(gl == m2, jnp.logical_not(g1))
    w1v = 1.0 / (1.0 + jnp.exp(m2 - m1))
    w2v = 1.0 - w1v
    gates = jnp.where(g1, w1v, 0.0) + jnp.where(g2, w2v, 0.0)
    acc = jnp.zeros((_NDST, _NT), jnp.float32)
    for ei in range(_NEXP):
        W1 = w1_ref[ei * _NT:(ei + 1) * _NT, :]
        h1 = jnp.maximum(
            jnp.dot(pooled, W1, preferred_element_type=jnp.float32), 0.0)
        W2 = w2_ref[ei * _DFF:(ei + 1) * _DFF, :]
        h2 = jnp.dot(h1, W2, preferred_element_type=jnp.float32)
        acc = acc + gates[:, ei:ei + 1] * h2
    out_ref[...] = 1.0 / (1.0 + jnp.exp(-acc))


def kernel(src_ids, etype, block_src, block_dst, e2t_ent, e2t_typ, t2c_typ,
           t2c_clu, e2c_ent, e2c_clu, entity_emb, type_emb, cluster_emb,
           relation, ln_g, ln_b, cl_w1, cl_b1, cl_w2, cl_b2, fc_w, fc_b,
           attn_a, gate_w, exp_w1, exp_w2):
    e2t_e, e2t_t = _light_gcn(entity_emb, type_emb, e2t_ent, e2t_typ)
    t2c_t, t2c_c = _light_gcn(type_emb, cluster_emb, t2c_typ, t2c_clu)
    e2c_e, e2c_c = _light_gcn(entity_emb, cluster_emb, e2c_ent, e2c_clu)
    ln = lambda x: _ln(x, ln_g, ln_b)
    node1 = jnp.concatenate([ln(e2t_e), ln(e2t_t), ln(t2c_c)], 0)
    node2 = jnp.concatenate([ln(e2c_e), ln(t2c_t), ln(e2c_c)], 0)
    src1 = node1[src_ids]; src2 = node2[src_ids]
    cl = _cl_loss(src1, src2, cl_w1, cl_b1, cl_w2, cl_b2)
    src = jnp.concatenate([src1, src2], -1)
    rel = relation[etype % _NR] * jnp.where(etype >= _NR, -1.0, 1.0)[:, None]
    reg = (jnp.sum(src ** 2) + jnp.sum(rel ** 2)) / 2.0
    aux = _CLW * cl + _DECAY * reg / _EB

    srcg = src[block_src]
    dst2 = block_dst.astype(jnp.int32).reshape(_EB, 1)
    fcb2 = fc_b.reshape(1, _NT)
    attn2 = attn_a.reshape(1, _H * _DH)

    p1, e = pl.pallas_call(
        _k1,
        grid=(_NBLK,),
        in_specs=[
            pl.BlockSpec((_BE, 2 * _D), lambda i: (i, 0)),
            pl.BlockSpec((_BE, 2 * _D), lambda i: (i, 0)),
            pl.BlockSpec((2 * _D, _NT), lambda i: (0, 0)),
            pl.BlockSpec((1, _NT), lambda i: (0, 0)),
            pl.BlockSpec((1, _H * _DH), lambda i: (0, 0)),
        ],
        out_specs=[
            pl.BlockSpec((_BE, _NT), lambda i: (i, 0)),
            pl.BlockSpec((_BE, _H), lambda i: (i, 0)),
        ],
        out_shape=[
            jax.ShapeDtypeStruct((_EB, _NT), jnp.float32),
            jax.ShapeDtypeStruct((_EB, _H), jnp.float32),
        ],
    )(srcg, rel, fc_w, fcb2, attn2)

    mx = pl.pallas_call(
        _k2,
        grid=(_NBLK,),
        in_specs=[
            pl.BlockSpec((_BE, _H), lambda i: (i, 0)),
            pl.BlockSpec((_BE, 1), lambda i: (i, 0)),
        ],
        out_specs=pl.BlockSpec((_H, _NDST), lambda i: (0, 0)),
        out_shape=jax.ShapeDtypeStruct((_H, _NDST), jnp.float32),
    )(e, dst2)

    den, num = pl.pallas_call(
        _k3,
        grid=(_NBLK,),
        in_specs=[
            pl.BlockSpec((_BE, _NT), lambda i: (i, 0)),
            pl.BlockSpec((_BE, _H), lambda i: (i, 0)),
            pl.BlockSpec((_BE, 1), lambda i: (i, 0)),
            pl.BlockSpec((_H, _NDST), lambda i: (0, 0)),
        ],
        out_specs=[
            pl.BlockSpec((_NDST, _H), lambda i: (0, 0)),
            pl.BlockSpec((_NDST, _NT), lambda i: (0, 0)),
        ],
        out_shape=[
            jax.ShapeDtypeStruct((_NDST, _H), jnp.float32),
            jax.ShapeDtypeStruct((_NDST, _NT), jnp.float32),
        ],
    )(p1, e, dst2, mx)

    predict = pl.pallas_call(
        _k4,
        grid=(1,),
        in_specs=[
            pl.BlockSpec((_NDST, _NT), lambda i: (0, 0)),
            pl.BlockSpec((_NDST, _H), lambda i: (0, 0)),
            pl.BlockSpec((_NT, _NEXP), lambda i: (0, 0)),
            pl.BlockSpec((_NEXP * _NT, _DFF), lambda i: (0, 0)),
            pl.BlockSpec((_NEXP * _DFF, _NT), lambda i: (0, 0)),
        ],
        out_specs=pl.BlockSpec((_NDST, _NT), lambda i: (0, 0)),
        out_shape=jax.ShapeDtypeStruct((_NDST, _NT), jnp.float32),
    )(num, den, gate_w, exp_w1.reshape(_NEXP * _NT, _DFF),
      exp_w2.reshape(_NEXP * _DFF, _NT))

    return predict, aux
